# baseline (speedup 1.0000x reference)
import functools

import jax
import jax.numpy as jnp
from jax import lax
from jax.experimental import pallas as pl
from jax.experimental.pallas import tpu as pltpu
from jax.experimental.pallas import tpu_sc as plsc

B, S, D = 4, 2048, 768
NUM_SPANS = 1024
NUM_WIDTH = 64
WIDTH_DIM = 128
OUT_D = 2 * D + WIDTH_DIM
TOTAL = B * NUM_SPANS

NC, NS, L = 2, 16, 16
NW = NC * NS
SPW = TOTAL // NW
CH = 64


def _body(seq_hbm, starts_hbm, ends_hbm, wt_hbm, out_hbm,
          sidx_v, eidx_v, widx_v, srow_v, erow_v, wrow_v, sem):
    wid = lax.axis_index("s") * NC + lax.axis_index("c")
    base = wid * SPW
    boff = (base // NUM_SPANS) * S
    for c in range(SPW // CH):
        cb = base + c * CH
        pltpu.sync_copy(starts_hbm.at[pl.ds(cb, CH)], sidx_v)
        pltpu.sync_copy(ends_hbm.at[pl.ds(cb, CH)], eidx_v)
        for i in range(CH // L):
            sl = pl.ds(i * L, L)
            s16 = sidx_v[sl]
            e16 = eidx_v[sl]
            widx_v[sl] = jnp.minimum(jnp.maximum(e16 - s16, 0), NUM_WIDTH - 1)
            sidx_v[sl] = s16 + boff
            eidx_v[sl] = e16 + boff
        g1 = pltpu.async_copy(seq_hbm.at[sidx_v], srow_v, sem)
        g2 = pltpu.async_copy(seq_hbm.at[eidx_v], erow_v, sem)
        g3 = pltpu.async_copy(wt_hbm.at[widx_v], wrow_v, sem)
        g1.wait()
        g2.wait()
        g3.wait()
        pltpu.sync_copy(srow_v, out_hbm.at[pl.ds(cb, CH), pl.ds(0, D)])
        pltpu.sync_copy(erow_v, out_hbm.at[pl.ds(cb, CH), pl.ds(D, D)])
        pltpu.sync_copy(wrow_v, out_hbm.at[pl.ds(cb, CH), pl.ds(2 * D, WIDTH_DIM)])


_sc_extract = functools.partial(
    pl.kernel,
    out_type=jax.ShapeDtypeStruct((TOTAL, OUT_D), jnp.float32),
    mesh=plsc.VectorSubcoreMesh(core_axis_name="c", subcore_axis_name="s"),
    compiler_params=pltpu.CompilerParams(use_tc_tiling_on_sc=False),
    scratch_types=[
        pltpu.VMEM((CH,), jnp.int32),
        pltpu.VMEM((CH,), jnp.int32),
        pltpu.VMEM((CH,), jnp.int32),
        pltpu.VMEM((CH, D), jnp.float32),
        pltpu.VMEM((CH, D), jnp.float32),
        pltpu.VMEM((CH, WIDTH_DIM), jnp.float32),
        pltpu.SemaphoreType.DMA,
    ],
)(_body)


def kernel(sequence_tensor, span_indices, width_table):
    seq = sequence_tensor.reshape(B * S, D)
    si = span_indices.astype(jnp.int32)
    starts = si[:, :, 0].reshape(TOTAL)
    ends = si[:, :, 1].reshape(TOTAL)
    out = _sc_extract(seq, starts, ends, width_table)
    return out.reshape(B, NUM_SPANS, OUT_D)


# speedup vs baseline: 1.3362x; 1.3362x over previous
import functools

import jax
import jax.numpy as jnp
from jax import lax
from jax.experimental import pallas as pl
from jax.experimental.pallas import tpu as pltpu
from jax.experimental.pallas import tpu_sc as plsc

B, S, D = 4, 2048, 768
NUM_SPANS = 1024
NUM_WIDTH = 64
WIDTH_DIM = 128
OUT_D = 2 * D + WIDTH_DIM
TOTAL = B * NUM_SPANS

NC, NS, L = 2, 16, 16
NW = NC * NS
SPW = TOTAL // NW
CH = 64
NSEM = 4
SUB = CH // NSEM


def _body(seq_hbm, starts_hbm, ends_hbm, wt_hbm, out_hbm,
          sidx_v, eidx_v, widx_v, srow_v, erow_v, wrow_v, *sems):
    wid = lax.axis_index("s") * NC + lax.axis_index("c")
    base = wid * SPW
    boff = (base // NUM_SPANS) * S
    for c in range(SPW // CH):
        cb = base + c * CH
        pltpu.sync_copy(starts_hbm.at[pl.ds(cb, CH)], sidx_v)
        pltpu.sync_copy(ends_hbm.at[pl.ds(cb, CH)], eidx_v)
        for i in range(CH // L):
            sl = pl.ds(i * L, L)
            s16 = sidx_v[sl]
            e16 = eidx_v[sl]
            widx_v[sl] = jnp.minimum(jnp.maximum(e16 - s16, 0), NUM_WIDTH - 1)
            sidx_v[sl] = s16 + boff
            eidx_v[sl] = e16 + boff
        hs = []
        for q in range(NSEM):
            qs = pl.ds(q * SUB, SUB)
            hs.append(pltpu.async_copy(
                seq_hbm.at[sidx_v.at[qs]], srow_v.at[qs], sems[q]))
            hs.append(pltpu.async_copy(
                seq_hbm.at[eidx_v.at[qs]], erow_v.at[qs], sems[(q + 1) % NSEM]))
        hs.append(pltpu.async_copy(wt_hbm.at[widx_v], wrow_v, sems[NSEM]))
        for h in hs:
            h.wait()
        pltpu.sync_copy(srow_v, out_hbm.at[pl.ds(cb, CH), pl.ds(0, D)])
        pltpu.sync_copy(erow_v, out_hbm.at[pl.ds(cb, CH), pl.ds(D, D)])
        pltpu.sync_copy(wrow_v, out_hbm.at[pl.ds(cb, CH), pl.ds(2 * D, WIDTH_DIM)])


_sc_extract = functools.partial(
    pl.kernel,
    out_type=jax.ShapeDtypeStruct((TOTAL, OUT_D), jnp.float32),
    mesh=plsc.VectorSubcoreMesh(core_axis_name="c", subcore_axis_name="s"),
    scratch_types=[
        pltpu.VMEM((CH,), jnp.int32),
        pltpu.VMEM((CH,), jnp.int32),
        pltpu.VMEM((CH,), jnp.int32),
        pltpu.VMEM((CH, D), jnp.float32),
        pltpu.VMEM((CH, D), jnp.float32),
        pltpu.VMEM((CH, WIDTH_DIM), jnp.float32),
        pltpu.SemaphoreType.DMA,
        pltpu.SemaphoreType.DMA,
        pltpu.SemaphoreType.DMA,
        pltpu.SemaphoreType.DMA,
        pltpu.SemaphoreType.DMA,
    ],
)(_body)


def kernel(sequence_tensor, span_indices, width_table):
    seq = sequence_tensor.reshape(B * S, D)
    si = span_indices.astype(jnp.int32)
    starts = si[:, :, 0].reshape(TOTAL)
    ends = si[:, :, 1].reshape(TOTAL)
    out = _sc_extract(seq, starts, ends, width_table)
    return out.reshape(B, NUM_SPANS, OUT_D)


# speedup vs baseline: 1.3654x; 1.0219x over previous
import functools

import jax
import jax.numpy as jnp
from jax import lax
from jax.experimental import pallas as pl
from jax.experimental.pallas import tpu as pltpu
from jax.experimental.pallas import tpu_sc as plsc

B, S, D = 4, 2048, 768
NUM_SPANS = 1024
NUM_WIDTH = 64
WIDTH_DIM = 128
OUT_D = 2 * D + WIDTH_DIM
TOTAL = B * NUM_SPANS

NC, NS, L = 2, 16, 16
NW = NC * NS
SPW = TOTAL // NW
CH = 64


def _body(seq_hbm, starts_hbm, ends_hbm, wt_hbm, out_hbm,
          sidx_v, eidx_v, widx_v, srow_v, erow_v, wrow_v, sem):
    wid = lax.axis_index("s") * NC + lax.axis_index("c")
    base = wid * SPW
    boff = (base // NUM_SPANS) * S
    for c in range(SPW // CH):
        cb = base + c * CH
        pltpu.sync_copy(starts_hbm.at[pl.ds(cb, CH)], sidx_v)
        pltpu.sync_copy(ends_hbm.at[pl.ds(cb, CH)], eidx_v)
        for i in range(CH // L):
            sl = pl.ds(i * L, L)
            s16 = sidx_v[sl]
            e16 = eidx_v[sl]
            widx_v[sl] = jnp.minimum(jnp.maximum(e16 - s16, 0), NUM_WIDTH - 1)
            sidx_v[sl] = s16 + boff
            eidx_v[sl] = e16 + boff
        g1 = pltpu.async_copy(seq_hbm.at[sidx_v], srow_v, sem)
        g2 = pltpu.async_copy(seq_hbm.at[eidx_v], erow_v, sem)
        g3 = pltpu.async_copy(wt_hbm.at[widx_v], wrow_v, sem)
        g1.wait()
        g2.wait()
        g3.wait()
        pltpu.sync_copy(srow_v, out_hbm.at[pl.ds(cb, CH), pl.ds(0, D)])
        pltpu.sync_copy(erow_v, out_hbm.at[pl.ds(cb, CH), pl.ds(D, D)])
        pltpu.sync_copy(wrow_v, out_hbm.at[pl.ds(cb, CH), pl.ds(2 * D, WIDTH_DIM)])


_sc_extract = functools.partial(
    pl.kernel,
    out_type=jax.ShapeDtypeStruct((TOTAL, OUT_D), jnp.float32),
    mesh=plsc.VectorSubcoreMesh(core_axis_name="c", subcore_axis_name="s"),
    scratch_types=[
        pltpu.VMEM((CH,), jnp.int32),
        pltpu.VMEM((CH,), jnp.int32),
        pltpu.VMEM((CH,), jnp.int32),
        pltpu.VMEM((CH, D), jnp.float32),
        pltpu.VMEM((CH, D), jnp.float32),
        pltpu.VMEM((CH, WIDTH_DIM), jnp.float32),
        pltpu.SemaphoreType.DMA,
    ],
)(_body)


def kernel(sequence_tensor, span_indices, width_table):
    seq = sequence_tensor.reshape(B * S, D)
    si = span_indices.astype(jnp.int32)
    starts = si[:, :, 0].reshape(TOTAL)
    ends = si[:, :, 1].reshape(TOTAL)
    out = _sc_extract(seq, starts, ends, width_table)
    return out.reshape(B, NUM_SPANS, OUT_D)
